# baseline (speedup 1.0000x reference)
import functools

import jax
import jax.numpy as jnp
from jax import lax
from jax.experimental import pallas as pl
from jax.experimental.pallas import tpu as pltpu
from jax.experimental.pallas import tpu_sc as plsc

_NUM_CORES = 2
_NUM_SUBCORES = 16
_NUM_WORKERS = _NUM_CORES * _NUM_SUBCORES
_CHUNK = 128


@functools.partial(jax.jit, static_argnames=("n_chunks", "embed"))
def _gather_sc(idx, table, *, n_chunks, embed):
    mesh = plsc.VectorSubcoreMesh(core_axis_name="c", subcore_axis_name="s")

    @functools.partial(
        pl.kernel,
        out_type=jax.ShapeDtypeStruct(
            (_NUM_WORKERS, n_chunks, _CHUNK, embed), jnp.float32
        ),
        mesh=mesh,
        compiler_params=pltpu.CompilerParams(use_tc_tiling_on_sc=False),
        scratch_types=[
            pltpu.VMEM((n_chunks, _CHUNK), jnp.int32),
            pltpu.VMEM((_CHUNK, embed), jnp.float32),
            pltpu.SemaphoreType.DMA,
        ],
    )
    def k(idx_hbm, table_hbm, out_hbm, idx_v, rows_v, sem):
        wid = lax.axis_index("s") * _NUM_CORES + lax.axis_index("c")
        pltpu.sync_copy(idx_hbm.at[wid], idx_v)

        def body(j, carry):
            pltpu.async_copy(table_hbm.at[idx_v.at[j]], rows_v, sem).wait()
            pltpu.sync_copy(rows_v, out_hbm.at[wid, j])
            return carry

        lax.fori_loop(0, n_chunks, body, 0)

    return k(idx, table)


def kernel(token_ids, table):
    b, l = token_ids.shape
    _, embed = table.shape
    n = b * l
    assert n % (_NUM_WORKERS * _CHUNK) == 0
    n_chunks = n // (_NUM_WORKERS * _CHUNK)
    idx = token_ids.astype(jnp.int32).reshape(_NUM_WORKERS, n_chunks, _CHUNK)
    out = _gather_sc(idx, table, n_chunks=n_chunks, embed=embed)
    return out.reshape(b, l, embed)


# speedup vs baseline: 1.0439x; 1.0439x over previous
import functools

import jax
import jax.numpy as jnp
from jax import lax
from jax.experimental import pallas as pl
from jax.experimental.pallas import tpu as pltpu
from jax.experimental.pallas import tpu_sc as plsc

_NUM_CORES = 2
_NUM_SUBCORES = 16
_NUM_WORKERS = _NUM_CORES * _NUM_SUBCORES
_CHUNK = 640
_NBUF = 3


@functools.partial(jax.jit, static_argnames=("n_chunks", "embed"))
def _gather_sc(idx, table, *, n_chunks, embed):
    mesh = plsc.VectorSubcoreMesh(core_axis_name="c", subcore_axis_name="s")

    @functools.partial(
        pl.kernel,
        out_type=jax.ShapeDtypeStruct(
            (_NUM_WORKERS, n_chunks, _CHUNK, embed), jnp.float32
        ),
        mesh=mesh,
        compiler_params=pltpu.CompilerParams(use_tc_tiling_on_sc=False),
        scratch_types=[
            pltpu.VMEM((n_chunks, _CHUNK), jnp.int32),
            pltpu.VMEM((_NBUF, _CHUNK, embed), jnp.float32),
            pltpu.SemaphoreType.DMA((_NBUF,)),
            pltpu.SemaphoreType.DMA((_NBUF,)),
        ],
    )
    def k(idx_hbm, table_hbm, out_hbm, idx_v, rows_v, gsem, osem):
        wid = lax.axis_index("s") * _NUM_CORES + lax.axis_index("c")
        pltpu.sync_copy(idx_hbm.at[wid], idx_v)

        gathers = [None] * n_chunks
        outs = [None] * n_chunks
        for g in range(n_chunks):
            b = g % _NBUF
            if g >= _NBUF:
                outs[g - _NBUF].wait()
            gathers[g] = pltpu.async_copy(
                table_hbm.at[idx_v.at[g]], rows_v.at[b], gsem.at[b]
            )
            if g >= 1:
                gathers[g - 1].wait()
                pb = (g - 1) % _NBUF
                outs[g - 1] = pltpu.async_copy(
                    rows_v.at[pb], out_hbm.at[wid, g - 1], osem.at[pb]
                )
        gathers[n_chunks - 1].wait()
        lb = (n_chunks - 1) % _NBUF
        outs[n_chunks - 1] = pltpu.async_copy(
            rows_v.at[lb], out_hbm.at[wid, n_chunks - 1], osem.at[lb]
        )
        for g in range(max(0, n_chunks - _NBUF), n_chunks):
            outs[g].wait()

    return k(idx, table)


def kernel(token_ids, table):
    b, l = token_ids.shape
    _, embed = table.shape
    n = b * l
    assert n % (_NUM_WORKERS * _CHUNK) == 0
    n_chunks = n // (_NUM_WORKERS * _CHUNK)
    idx = token_ids.astype(jnp.int32).reshape(_NUM_WORKERS, n_chunks, _CHUNK)
    out = _gather_sc(idx, table, n_chunks=n_chunks, embed=embed)
    return out.reshape(b, l, embed)


# speedup vs baseline: 1.1080x; 1.0614x over previous
import functools

import jax
import jax.numpy as jnp
from jax import lax
from jax.experimental import pallas as pl
from jax.experimental.pallas import tpu as pltpu
from jax.experimental.pallas import tpu_sc as plsc

_NUM_CORES = 2
_NUM_SUBCORES = 16
_NUM_WORKERS = _NUM_CORES * _NUM_SUBCORES
_CHUNK = 256
_NBUF = 3
_LANES = 128


@functools.partial(jax.jit, static_argnames=("n_chunks", "embed"))
def _gather_sc(idx, table_pad, *, n_chunks, embed):
    mesh = plsc.VectorSubcoreMesh(core_axis_name="c", subcore_axis_name="s")

    @functools.partial(
        pl.kernel,
        out_type=jax.ShapeDtypeStruct(
            (_NUM_WORKERS, n_chunks, _CHUNK, embed), jnp.float32
        ),
        mesh=mesh,
        compiler_params=pltpu.CompilerParams(use_tc_tiling_on_sc=False),
        scratch_types=[
            pltpu.VMEM((n_chunks, _CHUNK), jnp.int32),
            pltpu.VMEM((_NBUF, _CHUNK, _LANES), jnp.float32),
            pltpu.SemaphoreType.DMA((_NBUF,)),
            pltpu.SemaphoreType.DMA((_NBUF,)),
        ],
    )
    def k(idx_hbm, table_hbm, out_hbm, idx_v, rows_v, gsem, osem):
        wid = lax.axis_index("s") * _NUM_CORES + lax.axis_index("c")
        pltpu.sync_copy(idx_hbm.at[wid], idx_v)

        gathers = [None] * n_chunks
        outs = [None] * n_chunks

        def start_out(g):
            b = g % _NBUF
            return pltpu.async_copy(
                rows_v.at[b, slice(None), pl.ds(0, embed)],
                out_hbm.at[wid, g],
                osem.at[b],
            )

        for g in range(n_chunks):
            b = g % _NBUF
            if g >= _NBUF:
                outs[g - _NBUF].wait()
            gathers[g] = pltpu.async_copy(
                table_hbm.at[idx_v.at[g]], rows_v.at[b], gsem.at[b]
            )
            if g >= 1:
                gathers[g - 1].wait()
                outs[g - 1] = start_out(g - 1)
        gathers[n_chunks - 1].wait()
        outs[n_chunks - 1] = start_out(n_chunks - 1)
        for g in range(max(0, n_chunks - _NBUF), n_chunks):
            outs[g].wait()

    return k(idx, table_pad)


def kernel(token_ids, table):
    b, l = token_ids.shape
    _, embed = table.shape
    n = b * l
    assert n % (_NUM_WORKERS * _CHUNK) == 0
    n_chunks = n // (_NUM_WORKERS * _CHUNK)
    idx = token_ids.astype(jnp.int32).reshape(_NUM_WORKERS, n_chunks, _CHUNK)
    table_pad = jnp.pad(table, ((0, 0), (0, _LANES - embed)))
    out = _gather_sc(idx, table_pad, n_chunks=n_chunks, embed=embed)
    return out.reshape(b, l, embed)
